# speedup vs baseline: 1.6430x; 1.6430x over previous
import functools

import jax
import jax.numpy as jnp
from jax import lax
from jax.experimental import pallas as pl
from jax.experimental.pallas import tpu as pltpu
from jax.experimental.pallas import tpu_sc as plsc

V = 100000
E = 128
H = 512
B = 8
S = 64
T = B * S
G = 4 * H


def _sc_gather(table, idx_flat):
    info = plsc.get_sparse_core_info()
    nc, ns = info.num_cores, info.num_subcores
    nw = nc * ns
    bpw = T // nw
    mesh = plsc.VectorSubcoreMesh(core_axis_name="c", subcore_axis_name="s")

    @functools.partial(
        pl.kernel,
        mesh=mesh,
        out_type=jax.ShapeDtypeStruct((T, E), jnp.float32),
        scratch_types=[
            pltpu.VMEM((bpw,), jnp.int32),
            pltpu.VMEM((bpw, E), jnp.float32),
            pltpu.SemaphoreType.DMA,
        ],
    )
    def k(table_hbm, idx_hbm, out_hbm, idx_v, rows_v, sem):
        wid = lax.axis_index("s") * nc + lax.axis_index("c")
        base = wid * bpw
        pltpu.sync_copy(idx_hbm.at[pl.ds(base, bpw)], idx_v)
        pltpu.async_copy(table_hbm.at[idx_v], rows_v, sem).wait()
        pltpu.sync_copy(rows_v, out_hbm.at[pl.ds(base, bpw)])

    return k(table, idx_flat)


def _lstm_body(x_ref, wih0t, whh0t, b0, wih1t, whh1t, b1, wp1t, bp1,
               wp2t, bp2, out_ref, pre0_ref, hs_ref):
    pre0_ref[...] = (
        jnp.dot(x_ref[...], wih0t[...], preferred_element_type=jnp.float32)
        + b0[...]
    )

    def step(t, carry):
        h0, c0, h1, c1 = carry
        g0 = pre0_ref[pl.ds(t * B, B), :] + jnp.dot(
            h0, whh0t[...], preferred_element_type=jnp.float32)
        i0 = jax.nn.sigmoid(g0[:, 0:H])
        f0 = jax.nn.sigmoid(g0[:, H:2 * H])
        gg0 = jnp.tanh(g0[:, 2 * H:3 * H])
        o0 = jax.nn.sigmoid(g0[:, 3 * H:4 * H])
        c0 = f0 * c0 + i0 * gg0
        h0 = o0 * jnp.tanh(c0)

        g1 = (b1[...] + jnp.dot(h0, wih1t[...],
                                preferred_element_type=jnp.float32)
              + jnp.dot(h1, whh1t[...], preferred_element_type=jnp.float32))
        i1 = jax.nn.sigmoid(g1[:, 0:H])
        f1 = jax.nn.sigmoid(g1[:, H:2 * H])
        gg1 = jnp.tanh(g1[:, 2 * H:3 * H])
        o1 = jax.nn.sigmoid(g1[:, 3 * H:4 * H])
        c1 = f1 * c1 + i1 * gg1
        h1 = o1 * jnp.tanh(c1)
        hs_ref[pl.ds(t * B, B), :] = h1
        return h0, c0, h1, c1

    z = jnp.zeros((B, H), jnp.float32)
    lax.fori_loop(0, S, step, (z, z, z, z), unroll=False)

    p1 = jnp.tanh(
        jnp.dot(hs_ref[...], wp1t[...], preferred_element_type=jnp.float32)
        + bp1[...])
    out_ref[...] = (
        jnp.dot(p1, wp2t[...], preferred_element_type=jnp.float32) + bp2[...])


def _lstm_proj(x_tm, wih0t, whh0t, b0, wih1t, whh1t, b1, wp1t, bp1, wp2t,
               bp2, interpret=False):
    return pl.pallas_call(
        _lstm_body,
        out_shape=jax.ShapeDtypeStruct((T, E), jnp.float32),
        scratch_shapes=[
            pltpu.VMEM((T, G), jnp.float32),
            pltpu.VMEM((T, H), jnp.float32),
        ],
        interpret=interpret,
    )(x_tm, wih0t, whh0t, b0, wih1t, whh1t, b1, wp1t, bp1, wp2t, bp2)


_TV = 2048


def _logits_body(x_ref, emb_ref, gb_ref, out_ref):
    out_ref[...] = lax.dot_general(
        x_ref[...], emb_ref[...],
        (((1,), (1,)), ((), ())),
        preferred_element_type=jnp.float32,
    ) + gb_ref[...]


def _logits(x_bm, emb_table, gen_b2d, interpret=False):
    nv = pl.cdiv(V, _TV)
    return pl.pallas_call(
        _logits_body,
        grid=(nv,),
        in_specs=[
            pl.BlockSpec((T, E), lambda i: (0, 0)),
            pl.BlockSpec((_TV, E), lambda i: (i, 0)),
            pl.BlockSpec((1, _TV), lambda i: (0, i)),
        ],
        out_specs=pl.BlockSpec((T, _TV), lambda i: (0, i)),
        out_shape=jax.ShapeDtypeStruct((T, V), jnp.float32),
        interpret=interpret,
    )(x_bm, emb_table, gen_b2d)


def kernel(sentence, emb_table, W_ih0, W_hh0, b_ih0, b_hh0, W_ih1, W_hh1,
           b_ih1, b_hh1, W_p1, b_p1, W_p2, b_p2, gen_b):
    idx_tm = jnp.transpose(sentence).reshape(T).astype(jnp.int32)
    x_tm = _sc_gather(emb_table, idx_tm)

    out_tm = _lstm_proj(
        x_tm,
        W_ih0.T, W_hh0.T, (b_ih0 + b_hh0).reshape(1, G),
        W_ih1.T, W_hh1.T, (b_ih1 + b_hh1).reshape(1, G),
        W_p1.T, b_p1.reshape(1, H),
        W_p2.T, b_p2.reshape(1, E),
    )
    out_bm = out_tm.reshape(S, B, E).transpose(1, 0, 2).reshape(T, E)

    logits = _logits(out_bm, emb_table, gen_b.reshape(1, V))
    return logits.reshape(B, S, V)


# speedup vs baseline: 1.6954x; 1.0319x over previous
import functools

import jax
import jax.numpy as jnp
from jax import lax
from jax.experimental import pallas as pl
from jax.experimental.pallas import tpu as pltpu
from jax.experimental.pallas import tpu_sc as plsc

V = 100000
E = 128
H = 512
B = 8
S = 64
T = B * S
G = 4 * H


def _sc_gather(table, idx_flat):
    info = plsc.get_sparse_core_info()
    nc, ns = info.num_cores, info.num_subcores
    nw = nc * ns
    bpw = T // nw
    mesh = plsc.VectorSubcoreMesh(core_axis_name="c", subcore_axis_name="s")

    @functools.partial(
        pl.kernel,
        mesh=mesh,
        out_type=jax.ShapeDtypeStruct((T, E), jnp.float32),
        scratch_types=[
            pltpu.VMEM((bpw,), jnp.int32),
            pltpu.VMEM((bpw, E), jnp.float32),
            pltpu.SemaphoreType.DMA,
        ],
    )
    def k(table_hbm, idx_hbm, out_hbm, idx_v, rows_v, sem):
        wid = lax.axis_index("s") * nc + lax.axis_index("c")
        base = wid * bpw
        pltpu.sync_copy(idx_hbm.at[pl.ds(base, bpw)], idx_v)
        pltpu.async_copy(table_hbm.at[idx_v], rows_v, sem).wait()
        pltpu.sync_copy(rows_v, out_hbm.at[pl.ds(base, bpw)])

    return k(table, idx_flat)


def _lstm_body(x_ref, wih0t, whh0t, b0, wih1t, whh1t, b1, wp1t, bp1,
               wp2t, bp2, out_ref, pre0_ref, hs_ref):
    pre0_ref[...] = (
        jnp.dot(x_ref[...], wih0t[...], preferred_element_type=jnp.float32)
        + b0[...]
    )

    def step(t, carry):
        h0, c0, h1, c1 = carry
        g0 = pre0_ref[pl.ds(t * B, B), :] + jnp.dot(
            h0.astype(jnp.bfloat16), whh0t[...],
            preferred_element_type=jnp.float32)
        i0 = jax.nn.sigmoid(g0[:, 0:H])
        f0 = jax.nn.sigmoid(g0[:, H:2 * H])
        gg0 = jnp.tanh(g0[:, 2 * H:3 * H])
        o0 = jax.nn.sigmoid(g0[:, 3 * H:4 * H])
        c0 = f0 * c0 + i0 * gg0
        h0 = o0 * jnp.tanh(c0)

        g1 = (b1[...] + jnp.dot(h0.astype(jnp.bfloat16), wih1t[...],
                                preferred_element_type=jnp.float32)
              + jnp.dot(h1.astype(jnp.bfloat16), whh1t[...],
                        preferred_element_type=jnp.float32))
        i1 = jax.nn.sigmoid(g1[:, 0:H])
        f1 = jax.nn.sigmoid(g1[:, H:2 * H])
        gg1 = jnp.tanh(g1[:, 2 * H:3 * H])
        o1 = jax.nn.sigmoid(g1[:, 3 * H:4 * H])
        c1 = f1 * c1 + i1 * gg1
        h1 = o1 * jnp.tanh(c1)
        hs_ref[pl.ds(t * B, B), :] = h1
        return h0, c0, h1, c1

    z = jnp.zeros((B, H), jnp.float32)
    lax.fori_loop(0, S, step, (z, z, z, z), unroll=False)

    p1 = jnp.tanh(
        jnp.dot(hs_ref[...], wp1t[...], preferred_element_type=jnp.float32)
        + bp1[...])
    out_ref[...] = (
        jnp.dot(p1, wp2t[...], preferred_element_type=jnp.float32) + bp2[...])


def _lstm_proj(x_tm, wih0t, whh0t, b0, wih1t, whh1t, b1, wp1t, bp1, wp2t,
               bp2, interpret=False):
    return pl.pallas_call(
        _lstm_body,
        out_shape=jax.ShapeDtypeStruct((T, E), jnp.float32),
        scratch_shapes=[
            pltpu.VMEM((T, G), jnp.float32),
            pltpu.VMEM((T, H), jnp.float32),
        ],
        interpret=interpret,
    )(x_tm, wih0t, whh0t, b0, wih1t, whh1t, b1, wp1t, bp1, wp2t, bp2)


_TV = 2048


def _logits_body(x_ref, emb_ref, gb_ref, out_ref):
    out_ref[...] = lax.dot_general(
        x_ref[...], emb_ref[...],
        (((1,), (1,)), ((), ())),
        preferred_element_type=jnp.float32,
    ) + gb_ref[...]


def _logits(x_bm, emb_table, gen_b2d, interpret=False):
    nv = pl.cdiv(V, _TV)
    return pl.pallas_call(
        _logits_body,
        grid=(nv,),
        in_specs=[
            pl.BlockSpec((T, E), lambda i: (0, 0)),
            pl.BlockSpec((_TV, E), lambda i: (i, 0)),
            pl.BlockSpec((1, _TV), lambda i: (0, i)),
        ],
        out_specs=pl.BlockSpec((T, _TV), lambda i: (0, i)),
        out_shape=jax.ShapeDtypeStruct((T, V), jnp.float32),
        interpret=interpret,
    )(x_bm, emb_table, gen_b2d)


def kernel(sentence, emb_table, W_ih0, W_hh0, b_ih0, b_hh0, W_ih1, W_hh1,
           b_ih1, b_hh1, W_p1, b_p1, W_p2, b_p2, gen_b):
    idx_tm = jnp.transpose(sentence).reshape(T).astype(jnp.int32)
    x_tm = _sc_gather(emb_table, idx_tm)

    out_tm = _lstm_proj(
        x_tm,
        W_ih0.T, W_hh0.T.astype(jnp.bfloat16), (b_ih0 + b_hh0).reshape(1, G),
        W_ih1.T.astype(jnp.bfloat16), W_hh1.T.astype(jnp.bfloat16),
        (b_ih1 + b_hh1).reshape(1, G),
        W_p1.T, b_p1.reshape(1, H),
        W_p2.T, b_p2.reshape(1, E),
    )
    out_bm = out_tm.reshape(S, B, E).transpose(1, 0, 2).reshape(T, E)

    logits = _logits(out_bm, emb_table, gen_b.reshape(1, V))
    return logits.reshape(B, S, V)
